# Initial kernel scaffold; baseline (speedup 1.0000x reference)
import jax
import jax.numpy as jnp
from jax.experimental import pallas as pl


def kernel(images, features, box_regression, centerness, box_cls, dot_product_logits, positive_map):
    raise NotImplementedError("write your pallas kernel here")


# speedup vs baseline: 17.9420x; 17.9420x over previous
import math

import jax
import jax.numpy as jnp
from jax.experimental import pallas as pl

STRIDE = 8
ANCHOR_SIZE = 64
PRE_NMS_THRESH = 0.05
PRE_NMS_TOP_N = 1000
NMS_TH = 0.6
MAX_DET = 100
BBOX_XFORM_CLIP = math.log(1000.0 / 16.0)

N = 4096
C = 80
K = 1024
NC_TOTAL = N * C

_HI = jax.lax.Precision.HIGHEST
_DEF = jax.lax.Precision.DEFAULT


def _dot(a, b, dims, prec):
    return jax.lax.dot_general(a, b, (dims, ((), ())), precision=prec,
                               preferred_element_type=jnp.float32)


def _select_body(dp_ref, pm_ref, ctn_ref, sel_ref):
    f32, i32 = jnp.float32, jnp.int32

    pmv = pm_ref[...]
    pmn = pmv / jnp.maximum(jnp.sum(pmv, axis=-1, keepdims=True), 1e-6)
    logits = _dot(dp_ref[...], pmn, ((1,), (1,)), _DEF)
    ctn = jax.nn.sigmoid(ctn_ref[...])
    sc = jnp.sqrt(jax.nn.sigmoid(logits) * ctn)
    sc = jnp.where(sc > PRE_NMS_THRESH, sc, 0.0)
    bits = jax.lax.bitcast_convert_type(sc, i32)

    c0 = jnp.sum((bits > 0).astype(i32))

    def bit_step(i, u):
        b = 30 - i
        u_try = jnp.bitwise_or(u, jax.lax.shift_left(jnp.int32(1), b))
        cnt = jnp.sum((bits > u_try).astype(i32))
        return jnp.where(cnt >= PRE_NMS_TOP_N, u_try, u)

    u = jax.lax.fori_loop(0, 31, bit_step, jnp.int32(0))
    v_bits = jnp.where(c0 >= PRE_NMS_TOP_N, u + 1, jnp.int32(0))
    m_strict = bits > v_bits
    m_eq = bits == v_bits
    n_strict = jnp.sum(m_strict.astype(i32))
    needed = PRE_NMS_TOP_N - n_strict

    row_i = jax.lax.broadcasted_iota(i32, (N, C), 0)
    col_i = jax.lax.broadcasted_iota(i32, (N, C), 1)
    flat_i = row_i * C + col_i

    def idx_step(i, t):
        b = 19 - i
        t_try = jnp.bitwise_or(t, jax.lax.shift_left(jnp.int32(1), b))
        cnt = jnp.sum((m_eq & (flat_i < t_try)).astype(i32))
        return jnp.where(cnt < needed, t_try, t)

    t_lo = jax.lax.fori_loop(0, 20, idx_step, jnp.int32(0))
    t_star = jnp.where(needed > 0, t_lo + 1, jnp.int32(0))
    self_f = (m_strict | (m_eq & (flat_i < t_star))).astype(f32)

    cnt_row = jnp.sum(self_f, axis=1, keepdims=True)
    cnt2d = cnt_row.reshape(32, 128)
    lane_q = jax.lax.broadcasted_iota(i32, (128, 128), 0)
    lane_p = jax.lax.broadcasted_iota(i32, (128, 128), 1)
    u_le = (lane_q <= lane_p).astype(f32)
    incl = _dot(cnt2d, u_le, ((1,), (0,)), _HI)
    tot = incl[:, 127:128]
    c32_q = jax.lax.broadcasted_iota(i32, (32, 32), 1)
    c32_p = jax.lax.broadcasted_iota(i32, (32, 32), 0)
    l32 = (c32_q < c32_p).astype(f32)
    off = _dot(l32, tot, ((1,), (0,)), _HI)
    rowpref = off + incl - cnt2d

    p_col = jax.lax.broadcasted_iota(i32, (K, 1), 0).astype(f32)
    r_cnt = jnp.zeros((K, 1), f32)
    for i in range(32):
        r_cnt = r_cnt + jnp.sum(
            (rowpref[i:i + 1, :] <= p_col).astype(f32), axis=1, keepdims=True)
    r_p = r_cnt - 1.0
    r_p_i = r_p.astype(i32)

    selrows = jnp.zeros((K, C), f32)
    scrows = jnp.zeros((K, C), f32)
    pref_r = jnp.zeros((K, 1), f32)
    for i in range(4):
        ch = jax.lax.broadcasted_iota(i32, (K, K), 1) + (i * K)
        oh = (ch == r_p_i).astype(f32)
        lt = (ch < r_p_i).astype(f32)
        selrows = selrows + _dot(oh, self_f[i * K:(i + 1) * K, :], ((1,), (0,)), _HI)
        scrows = scrows + _dot(oh, sc[i * K:(i + 1) * K, :], ((1,), (0,)), _HI)
        pref_r = pref_r + _dot(lt, cnt_row[i * K:(i + 1) * K, :], ((1,), (0,)), _HI)

    q_p = p_col - pref_r
    c_lt = jax.lax.broadcasted_iota(i32, (C, C), 0)
    c_gt = jax.lax.broadcasted_iota(i32, (C, C), 1)
    lstrict_c = (c_lt < c_gt).astype(f32)
    wpref = _dot(selrows, lstrict_c, ((1,), (0,)), _HI)
    colmatch = selrows * (wpref == q_p).astype(f32)
    col_iota = jax.lax.broadcasted_iota(i32, (K, C), 1).astype(f32)
    c_p = jnp.sum(colmatch * col_iota, axis=1, keepdims=True)
    score_p = jnp.sum(colmatch * scrows, axis=1, keepdims=True)

    real = p_col < float(PRE_NMS_TOP_N)
    score_p = jnp.where(real, score_p, 0.0)
    box_p = jnp.where(real, r_p, 0.0)
    cls_p = jnp.where(real, c_p, 0.0)

    def to_row(x):
        return jnp.swapaxes(x, 0, 1)

    k_col = jnp.where(real, score_p, -1.0)
    k_row = to_row(k_col)
    q_idx = jax.lax.broadcasted_iota(i32, (K, K), 1)
    p_idx = jax.lax.broadcasted_iota(i32, (K, K), 0)
    better = (k_row > k_col) | ((k_row == k_col) & (q_idx < p_idx))
    rank = jnp.sum(better.astype(f32), axis=1, keepdims=True)
    rank_row = to_row(rank)
    onehot_s = (rank_row == p_col).astype(f32)

    s_score = _dot(onehot_s, score_p, ((1,), (0,)), _HI)
    s_box = _dot(onehot_s, box_p, ((1,), (0,)), _HI)
    s_cls = _dot(onehot_s, cls_p, ((1,), (0,)), _HI)
    sel_ref[...] = jnp.concatenate(
        [s_score, s_box, s_cls, jnp.zeros((K, 5), f32)], axis=1)


def _nms_body(sel_ref, br_ref, out_ref):
    f32, i32 = jnp.float32, jnp.int32
    s_score = sel_ref[:, 0:1]
    s_box = sel_ref[:, 1:2]
    s_cls = sel_ref[:, 2:3]
    s_box_i = s_box.astype(i32)
    p_col = jax.lax.broadcasted_iota(i32, (K, 1), 0).astype(f32)
    real = p_col < float(PRE_NMS_TOP_N)
    q_idx = jax.lax.broadcasted_iota(i32, (K, K), 1)
    p_idx = jax.lax.broadcasted_iota(i32, (K, K), 0)

    def to_row(x):
        return jnp.swapaxes(x, 0, 1)

    deltas = jnp.zeros((K, 4), f32)
    for i in range(4):
        ch = jax.lax.broadcasted_iota(i32, (K, K), 1) + (i * K)
        ohb = (ch == s_box_i).astype(f32)
        deltas = deltas + _dot(ohb, br_ref[i * K:(i + 1) * K, :], ((1,), (0,)), _HI)

    by = jnp.floor(s_box * (1.0 / 64.0))
    bx = s_box - 64.0 * by
    ax = bx * float(STRIDE)
    ay = by * float(STRIDE)
    widths = jnp.full((K, 1), 64.0, f32)
    ctr_x = ax + 0.5
    ctr_y = ay + 0.5
    dx = deltas[:, 0:1] / 10.0
    dy = deltas[:, 1:2] / 10.0
    dw = jnp.minimum(deltas[:, 2:3] / 5.0, BBOX_XFORM_CLIP)
    dh = jnp.minimum(deltas[:, 3:4] / 5.0, BBOX_XFORM_CLIP)
    pcx = dx * widths + ctr_x
    pcy = dy * widths + ctr_y
    pw = jnp.exp(dw) * widths
    ph = jnp.exp(dh) * widths
    x1 = pcx - 0.5 * (pw - 1.0)
    y1 = pcy - 0.5 * (ph - 1.0)
    x2 = pcx + 0.5 * (pw - 1.0)
    y2 = pcy + 0.5 * (ph - 1.0)
    x1 = jnp.clip(x1, 0.0, 511.0)
    y1 = jnp.clip(y1, 0.0, 511.0)
    x2 = jnp.clip(x2, 0.0, 511.0)
    y2 = jnp.clip(y2, 0.0, 511.0)

    offv = s_cls * 513.0
    pad_off = jnp.where(real, 0.0, -1.0e7 - p_col * 1.0e3)
    nx1 = x1 + offv + pad_off
    ny1 = y1 + offv + pad_off
    nx2 = jnp.where(real, x2 + offv, nx1 - 1.0)
    ny2 = jnp.where(real, y2 + offv, ny1 - 1.0)

    areas = (nx2 - nx1 + 1.0) * (ny2 - ny1 + 1.0)
    xx1 = jnp.maximum(nx1, to_row(nx1))
    yy1 = jnp.maximum(ny1, to_row(ny1))
    xx2 = jnp.minimum(nx2, to_row(nx2))
    yy2 = jnp.minimum(ny2, to_row(ny2))
    w = jnp.maximum(xx2 - xx1 + 1.0, 0.0)
    h = jnp.maximum(yy2 - yy1 + 1.0, 0.0)
    inter = w * h
    iou = inter / (areas + to_row(areas) - inter)
    a_sup = ((iou > NMS_TH) & (p_idx < q_idx)).astype(f32)

    def cond(carry):
        return carry[1]

    def body(carry):
        keep, _ = carry
        sup = _dot(keep, a_sup, ((1,), (0,)), _DEF)
        new = jnp.where(sup > 0.0, 0.0, 1.0)
        return new, jnp.any(new != keep)

    keep0 = jnp.ones((1, K), f32)
    keep_row, _ = jax.lax.while_loop(cond, body, (keep0, jnp.bool_(True)))

    keep_col = jnp.swapaxes(keep_row, 0, 1)

    valid = keep_col * real.astype(f32)
    masked = jnp.where(valid > 0.0, s_score, -1.0e9)
    m_row = to_row(masked)
    better2 = (m_row > masked) | ((m_row == masked) & (q_idx < p_idx))
    rank2 = jnp.sum(better2.astype(f32), axis=1, keepdims=True)
    rank2_row = to_row(rank2)

    boxes4 = jnp.concatenate([x1, y1, x2, y2], axis=1) * valid
    score_out = jnp.where(valid > 0.0, s_score, 0.0)

    p128 = jax.lax.broadcasted_iota(i32, (128, K), 0).astype(f32)
    onehot_f = (rank2_row == p128).astype(f32)
    out4 = _dot(onehot_f, boxes4, ((1,), (0,)), _HI)
    outs = _dot(onehot_f, score_out, ((1,), (0,)), _HI)
    out_ref[...] = jnp.concatenate(
        [out4, outs, jnp.zeros((128, 3), f32)], axis=1)


def kernel(images, features, box_regression, centerness, box_cls,
           dot_product_logits, positive_map):
    del images, features, box_cls
    dp = dot_product_logits.reshape(N, 256)
    ctn = centerness.reshape(N, 1)
    br = jnp.transpose(box_regression.reshape(4, N), (1, 0))
    sel = pl.pallas_call(
        _select_body,
        out_shape=jax.ShapeDtypeStruct((K, 8), jnp.float32),
    )(dp, positive_map, ctn)
    out = pl.pallas_call(
        _nms_body,
        out_shape=jax.ShapeDtypeStruct((128, 8), jnp.float32),
    )(sel, br)
    return out[:MAX_DET, :5].reshape(1, MAX_DET, 5)


# speedup vs baseline: 32.1931x; 1.7943x over previous
import math

import jax
import jax.numpy as jnp
from jax.experimental import pallas as pl

STRIDE = 8
ANCHOR_SIZE = 64
PRE_NMS_THRESH = 0.05
PRE_NMS_TOP_N = 1000
NMS_TH = 0.6
MAX_DET = 100
BBOX_XFORM_CLIP = math.log(1000.0 / 16.0)

N = 4096
C = 80
K = 1024
NC_TOTAL = N * C

_HI = jax.lax.Precision.HIGHEST
_DEF = jax.lax.Precision.DEFAULT


def _dot(a, b, dims, prec):
    return jax.lax.dot_general(a, b, (dims, ((), ())), precision=prec,
                               preferred_element_type=jnp.float32)


def _select_body(dp_ref, pm_ref, ctn_ref, br_ref, sel_ref):
    f32, i32 = jnp.float32, jnp.int32

    pmv = pm_ref[...]
    pmn = pmv / jnp.maximum(jnp.sum(pmv, axis=-1, keepdims=True), 1e-6)
    logits = _dot(dp_ref[...], pmn, ((1,), (1,)), _DEF)
    ctn = jax.nn.sigmoid(ctn_ref[...])
    sc = jnp.sqrt(jax.nn.sigmoid(logits) * ctn)
    sc = jnp.where(sc > PRE_NMS_THRESH, sc, 0.0)
    bits = jax.lax.bitcast_convert_type(sc, i32)

    c0 = jnp.sum((bits > 0).astype(i32))

    def bit_step(i, u):
        b = 29 - i
        u_try = jnp.bitwise_or(u, jax.lax.shift_left(jnp.int32(1), b))
        cnt = jnp.sum((bits > u_try).astype(i32))
        return jnp.where(cnt >= PRE_NMS_TOP_N, u_try, u)

    u = jax.lax.fori_loop(0, 30, bit_step, jnp.int32(0))
    v_bits = jnp.where(c0 >= PRE_NMS_TOP_N, u + 1, jnp.int32(0))
    m_strict = bits > v_bits
    m_eq = bits == v_bits
    n_strict = jnp.sum(m_strict.astype(i32))
    needed = PRE_NMS_TOP_N - n_strict

    row_i = jax.lax.broadcasted_iota(i32, (N, C), 0)
    col_i = jax.lax.broadcasted_iota(i32, (N, C), 1)
    flat_i = row_i * C + col_i

    def idx_step(i, t):
        b = 18 - i
        t_try = jnp.bitwise_or(t, jax.lax.shift_left(jnp.int32(1), b))
        cnt = jnp.sum((m_eq & (flat_i < t_try)).astype(i32))
        return jnp.where(cnt < needed, t_try, t)

    t_lo = jax.lax.fori_loop(0, 19, idx_step, jnp.int32(0))
    t_star = jnp.where(needed > 0, t_lo + 1, jnp.int32(0))
    self_f = (m_strict | (m_eq & (flat_i < t_star))).astype(f32)

    cnt_row = jnp.sum(self_f, axis=1, keepdims=True)
    cnt2d = cnt_row.reshape(32, 128)
    lane_q = jax.lax.broadcasted_iota(i32, (128, 128), 0)
    lane_p = jax.lax.broadcasted_iota(i32, (128, 128), 1)
    u_le = (lane_q <= lane_p).astype(f32)
    incl = _dot(cnt2d, u_le, ((1,), (0,)), _HI)
    tot = incl[:, 127:128]
    c32_q = jax.lax.broadcasted_iota(i32, (32, 32), 1)
    c32_p = jax.lax.broadcasted_iota(i32, (32, 32), 0)
    l32 = (c32_q < c32_p).astype(f32)
    off = _dot(l32, tot, ((1,), (0,)), _HI)
    rowpref = off + incl - cnt2d

    p_col = jax.lax.broadcasted_iota(i32, (K, 1), 0).astype(f32)
    r_cnt = jnp.zeros((K, 1), f32)
    pref_r = jnp.full((K, 1), -1.0, f32)
    for i in range(32):
        row = rowpref[i:i + 1, :]
        m = row <= p_col
        r_cnt = r_cnt + jnp.sum(m.astype(f32), axis=1, keepdims=True)
        pref_r = jnp.maximum(
            pref_r, jnp.max(jnp.where(m, row, -1.0), axis=1, keepdims=True))
    r_p = r_cnt - 1.0
    r_p_i = r_p.astype(i32)

    m16 = jnp.int32(-65536)
    def planes(x):
        xhi = jax.lax.bitcast_convert_type(
            jnp.bitwise_and(jax.lax.bitcast_convert_type(x, i32), m16), f32)
        xlo = x - xhi
        xmid = jax.lax.bitcast_convert_type(
            jnp.bitwise_and(jax.lax.bitcast_convert_type(xlo, i32), m16), f32)
        return xhi, xmid, xlo - xmid

    schi, scmid, sclo = planes(sc)
    brv = br_ref[...]
    brhi, brmid, brlo = planes(brv)
    payload = jnp.concatenate(
        [self_f, schi, scmid, sclo, brhi, brmid, brlo], axis=1)

    gath = jnp.zeros((K, 332), f32)
    for i in range(4):
        ch = jax.lax.broadcasted_iota(i32, (K, K), 1) + (i * K)
        oh = (ch == r_p_i).astype(f32)
        gath = gath + _dot(oh, payload[i * K:(i + 1) * K, :], ((1,), (0,)), _DEF)
    selrows = gath[:, 0:C]
    scrows = (gath[:, C:2 * C] + gath[:, 2 * C:3 * C]) + gath[:, 3 * C:4 * C]
    br_p = (gath[:, 4 * C:4 * C + 4] + gath[:, 4 * C + 4:4 * C + 8]) \
        + gath[:, 4 * C + 8:4 * C + 12]

    q_p = p_col - pref_r
    c_lt = jax.lax.broadcasted_iota(i32, (C, C), 0)
    c_gt = jax.lax.broadcasted_iota(i32, (C, C), 1)
    lstrict_c = (c_lt < c_gt).astype(f32)
    wpref = _dot(selrows, lstrict_c, ((1,), (0,)), _DEF)
    colmatch = selrows * (wpref == q_p).astype(f32)
    col_iota = jax.lax.broadcasted_iota(i32, (K, C), 1).astype(f32)
    c_p = jnp.sum(colmatch * col_iota, axis=1, keepdims=True)
    score_p = jnp.sum(colmatch * scrows, axis=1, keepdims=True)

    real = p_col < float(PRE_NMS_TOP_N)
    score_p = jnp.where(real, score_p, 0.0)
    box_p = jnp.where(real, r_p, 0.0)
    cls_p = jnp.where(real, c_p, 0.0)
    br_p = jnp.where(real, br_p, 0.0)

    def to_row(x):
        return jnp.swapaxes(x, 0, 1)

    k_col = jnp.where(real, score_p, -1.0)
    k_row = to_row(k_col)
    q_idx = jax.lax.broadcasted_iota(i32, (K, K), 1)
    p_idx = jax.lax.broadcasted_iota(i32, (K, K), 0)
    better = (k_row > k_col) | ((k_row == k_col) & (q_idx < p_idx))
    rank = jnp.sum(better.astype(f32), axis=1, keepdims=True)
    rank_row = to_row(rank)
    onehot_s = (rank_row == p_col).astype(f32)

    pay2 = jnp.concatenate([score_p, box_p, cls_p, br_p], axis=1)
    s_pay = _dot(onehot_s, pay2, ((1,), (0,)), _HI)
    sel_ref[...] = jnp.concatenate([s_pay, jnp.zeros((K, 1), f32)], axis=1)


def _nms_body(sel_ref, out_ref):
    f32, i32 = jnp.float32, jnp.int32
    s_score = sel_ref[:, 0:1]
    s_box = sel_ref[:, 1:2]
    s_cls = sel_ref[:, 2:3]
    deltas = sel_ref[:, 3:7]
    p_col = jax.lax.broadcasted_iota(i32, (K, 1), 0).astype(f32)
    real = p_col < float(PRE_NMS_TOP_N)
    q_idx = jax.lax.broadcasted_iota(i32, (K, K), 1)
    p_idx = jax.lax.broadcasted_iota(i32, (K, K), 0)

    def to_row(x):
        return jnp.swapaxes(x, 0, 1)

    by = jnp.floor(s_box * (1.0 / 64.0))
    bx = s_box - 64.0 * by
    ax = bx * float(STRIDE)
    ay = by * float(STRIDE)
    widths = jnp.full((K, 1), 64.0, f32)
    ctr_x = ax + 0.5
    ctr_y = ay + 0.5
    dx = deltas[:, 0:1] / 10.0
    dy = deltas[:, 1:2] / 10.0
    dw = jnp.minimum(deltas[:, 2:3] / 5.0, BBOX_XFORM_CLIP)
    dh = jnp.minimum(deltas[:, 3:4] / 5.0, BBOX_XFORM_CLIP)
    pcx = dx * widths + ctr_x
    pcy = dy * widths + ctr_y
    pw = jnp.exp(dw) * widths
    ph = jnp.exp(dh) * widths
    x1 = pcx - 0.5 * (pw - 1.0)
    y1 = pcy - 0.5 * (ph - 1.0)
    x2 = pcx + 0.5 * (pw - 1.0)
    y2 = pcy + 0.5 * (ph - 1.0)
    x1 = jnp.clip(x1, 0.0, 511.0)
    y1 = jnp.clip(y1, 0.0, 511.0)
    x2 = jnp.clip(x2, 0.0, 511.0)
    y2 = jnp.clip(y2, 0.0, 511.0)

    offv = s_cls * 513.0
    pad_off = jnp.where(real, 0.0, -1.0e7 - p_col * 1.0e3)
    nx1 = x1 + offv + pad_off
    ny1 = y1 + offv + pad_off
    nx2 = jnp.where(real, x2 + offv, nx1 - 1.0)
    ny2 = jnp.where(real, y2 + offv, ny1 - 1.0)

    areas = (nx2 - nx1 + 1.0) * (ny2 - ny1 + 1.0)
    xx1 = jnp.maximum(nx1, to_row(nx1))
    yy1 = jnp.maximum(ny1, to_row(ny1))
    xx2 = jnp.minimum(nx2, to_row(nx2))
    yy2 = jnp.minimum(ny2, to_row(ny2))
    w = jnp.maximum(xx2 - xx1 + 1.0, 0.0)
    h = jnp.maximum(yy2 - yy1 + 1.0, 0.0)
    inter = w * h
    iou = inter / (areas + to_row(areas) - inter)
    a_sup = ((iou > NMS_TH) & (p_idx < q_idx)).astype(f32)

    def cond(carry):
        return carry[1]

    def sweep(keep):
        sup = _dot(keep, a_sup, ((1,), (0,)), _DEF)
        return jnp.where(sup > 0.0, 0.0, 1.0)

    def body(carry):
        keep, _ = carry
        new = sweep(sweep(keep))
        return new, jnp.any(new != keep)

    keep0 = jnp.ones((1, K), f32)
    keep_row, _ = jax.lax.while_loop(cond, body, (keep0, jnp.bool_(True)))

    keep_col = jnp.swapaxes(keep_row, 0, 1)

    valid = keep_col * real.astype(f32)
    masked = jnp.where(valid > 0.0, s_score, -1.0e9)
    m_row = to_row(masked)
    better2 = (m_row > masked) | ((m_row == masked) & (q_idx < p_idx))
    rank2 = jnp.sum(better2.astype(f32), axis=1, keepdims=True)
    rank2_row = to_row(rank2)

    boxes4 = jnp.concatenate([x1, y1, x2, y2], axis=1) * valid
    score_out = jnp.where(valid > 0.0, s_score, 0.0)

    p128 = jax.lax.broadcasted_iota(i32, (128, K), 0).astype(f32)
    onehot_f = (rank2_row == p128).astype(f32)
    out4 = _dot(onehot_f, boxes4, ((1,), (0,)), _HI)
    outs = _dot(onehot_f, score_out, ((1,), (0,)), _HI)
    out_ref[...] = jnp.concatenate(
        [out4, outs, jnp.zeros((128, 3), f32)], axis=1)


def kernel(images, features, box_regression, centerness, box_cls,
           dot_product_logits, positive_map):
    del images, features, box_cls
    dp = dot_product_logits.reshape(N, 256)
    ctn = centerness.reshape(N, 1)
    br = jnp.transpose(box_regression.reshape(4, N), (1, 0))
    sel = pl.pallas_call(
        _select_body,
        out_shape=jax.ShapeDtypeStruct((K, 8), jnp.float32),
    )(dp, positive_map, ctn, br)
    out = pl.pallas_call(
        _nms_body,
        out_shape=jax.ShapeDtypeStruct((128, 8), jnp.float32),
    )(sel)
    return out[:MAX_DET, :5].reshape(1, MAX_DET, 5)


# speedup vs baseline: 32.2386x; 1.0014x over previous
import math

import jax
import jax.numpy as jnp
from jax.experimental import pallas as pl

STRIDE = 8
ANCHOR_SIZE = 64
PRE_NMS_THRESH = 0.05
PRE_NMS_TOP_N = 1000
NMS_TH = 0.6
MAX_DET = 100
BBOX_XFORM_CLIP = math.log(1000.0 / 16.0)

N = 4096
C = 80
K = 1024
NC_TOTAL = N * C

_HI = jax.lax.Precision.HIGHEST
_DEF = jax.lax.Precision.DEFAULT


def _dot(a, b, dims, prec):
    return jax.lax.dot_general(a, b, (dims, ((), ())), precision=prec,
                               preferred_element_type=jnp.float32)


def _select_body(dp_ref, pm_ref, ctn_ref, br_ref, sel_ref):
    f32, i32 = jnp.float32, jnp.int32

    pmv = pm_ref[...]
    pmn = pmv / jnp.maximum(jnp.sum(pmv, axis=-1, keepdims=True), 1e-6)
    logits = _dot(dp_ref[...], pmn, ((1,), (1,)), _DEF)
    ctn = jax.nn.sigmoid(ctn_ref[...])
    sc = jnp.sqrt(jax.nn.sigmoid(logits) * ctn)
    sc = jnp.where(sc > PRE_NMS_THRESH, sc, 0.0)
    bits = jax.lax.bitcast_convert_type(sc, i32)
    bits_s = bits

    c0 = jnp.sum((bits_s > 0).astype(i32))

    def bit_step(i, u):
        b = 29 - i
        u_try = jnp.bitwise_or(u, jax.lax.shift_left(jnp.int32(1), b))
        cnt = jnp.sum((bits_s > u_try).astype(i32))
        return jnp.where(cnt >= PRE_NMS_TOP_N, u_try, u)

    u = jax.lax.fori_loop(0, 30, bit_step, jnp.int32(0))
    v_bits = jnp.where(c0 >= PRE_NMS_TOP_N, u + 1, jnp.int32(0))
    m_eq_s = bits_s == v_bits
    n_strict = jnp.sum((bits_s > v_bits).astype(i32))
    needed = PRE_NMS_TOP_N - n_strict

    flat_s = (jax.lax.broadcasted_iota(i32, (N, C), 0) * C
              + jax.lax.broadcasted_iota(i32, (N, C), 1))

    def idx_step(i, t):
        b = 18 - i
        t_try = jnp.bitwise_or(t, jax.lax.shift_left(jnp.int32(1), b))
        cnt = jnp.sum((m_eq_s & (flat_s < t_try)).astype(i32))
        return jnp.where(cnt < needed, t_try, t)

    t_lo = jax.lax.fori_loop(0, 19, idx_step, jnp.int32(0))
    t_star = jnp.where(needed > 0, t_lo + 1, jnp.int32(0))
    row_i = jax.lax.broadcasted_iota(i32, (N, C), 0)
    col_i = jax.lax.broadcasted_iota(i32, (N, C), 1)
    flat_i = row_i * C + col_i
    self_f = ((bits > v_bits)
              | ((bits == v_bits) & (flat_i < t_star))).astype(f32)

    cnt_row = jnp.sum(self_f, axis=1, keepdims=True)
    cnt2d = cnt_row.reshape(32, 128)
    lane_q = jax.lax.broadcasted_iota(i32, (128, 128), 0)
    lane_p = jax.lax.broadcasted_iota(i32, (128, 128), 1)
    u_le = (lane_q <= lane_p).astype(f32)
    incl = _dot(cnt2d, u_le, ((1,), (0,)), _HI)
    tot = incl[:, 127:128]
    c32_q = jax.lax.broadcasted_iota(i32, (32, 32), 1)
    c32_p = jax.lax.broadcasted_iota(i32, (32, 32), 0)
    l32 = (c32_q < c32_p).astype(f32)
    off = _dot(l32, tot, ((1,), (0,)), _HI)
    rowpref = off + incl - cnt2d

    p_col = jax.lax.broadcasted_iota(i32, (K, 1), 0).astype(f32)
    r_cnt = jnp.zeros((K, 1), f32)
    pref_r = jnp.full((K, 1), -1.0, f32)
    for i in range(32):
        row = rowpref[i:i + 1, :]
        m = row <= p_col
        r_cnt = r_cnt + jnp.sum(m.astype(f32), axis=1, keepdims=True)
        pref_r = jnp.maximum(
            pref_r, jnp.max(jnp.where(m, row, -1.0), axis=1, keepdims=True))
    r_p = r_cnt - 1.0
    r_p_i = r_p.astype(i32)

    m16 = jnp.int32(-65536)
    def planes(x):
        xhi = jax.lax.bitcast_convert_type(
            jnp.bitwise_and(jax.lax.bitcast_convert_type(x, i32), m16), f32)
        xlo = x - xhi
        xmid = jax.lax.bitcast_convert_type(
            jnp.bitwise_and(jax.lax.bitcast_convert_type(xlo, i32), m16), f32)
        return xhi, xmid, xlo - xmid

    schi, scmid, sclo = planes(sc)
    brv = br_ref[...]
    brhi, brmid, brlo = planes(brv)
    payload = jnp.concatenate(
        [self_f, schi, scmid, sclo, brhi, brmid, brlo], axis=1)

    gath = jnp.zeros((K, 332), f32)
    for i in range(4):
        ch = jax.lax.broadcasted_iota(i32, (K, K), 1) + (i * K)
        oh = (ch == r_p_i).astype(f32)
        gath = gath + _dot(oh, payload[i * K:(i + 1) * K, :], ((1,), (0,)), _DEF)
    selrows = gath[:, 0:C]
    scrows = (gath[:, C:2 * C] + gath[:, 2 * C:3 * C]) + gath[:, 3 * C:4 * C]
    br_p = (gath[:, 4 * C:4 * C + 4] + gath[:, 4 * C + 4:4 * C + 8]) \
        + gath[:, 4 * C + 8:4 * C + 12]

    q_p = p_col - pref_r
    c_lt = jax.lax.broadcasted_iota(i32, (C, C), 0)
    c_gt = jax.lax.broadcasted_iota(i32, (C, C), 1)
    lstrict_c = (c_lt < c_gt).astype(f32)
    wpref = _dot(selrows, lstrict_c, ((1,), (0,)), _DEF)
    colmatch = selrows * (wpref == q_p).astype(f32)
    col_iota = jax.lax.broadcasted_iota(i32, (K, C), 1).astype(f32)
    c_p = jnp.sum(colmatch * col_iota, axis=1, keepdims=True)
    score_p = jnp.sum(colmatch * scrows, axis=1, keepdims=True)

    real = p_col < float(PRE_NMS_TOP_N)
    score_p = jnp.where(real, score_p, 0.0)
    box_p = jnp.where(real, r_p, 0.0)
    cls_p = jnp.where(real, c_p, 0.0)
    br_p = jnp.where(real, br_p, 0.0)

    def to_row(x):
        return jnp.swapaxes(x, 0, 1)

    k_col = jnp.where(real, score_p, -1.0)
    k_row = to_row(k_col)
    q_idx = jax.lax.broadcasted_iota(i32, (K, K), 1)
    p_idx = jax.lax.broadcasted_iota(i32, (K, K), 0)
    better = (k_row > k_col) | ((k_row == k_col) & (q_idx < p_idx))
    rank = jnp.sum(better.astype(f32), axis=1, keepdims=True)
    rank_row = to_row(rank)
    onehot_s = (rank_row == p_col).astype(f32)

    pay2 = jnp.concatenate([score_p, box_p, cls_p, br_p], axis=1)
    s_pay = _dot(onehot_s, pay2, ((1,), (0,)), _HI)
    sel_ref[...] = jnp.concatenate([s_pay, jnp.zeros((K, 1), f32)], axis=1)


def _nms_body(sel_ref, out_ref):
    f32, i32 = jnp.float32, jnp.int32
    s_score = sel_ref[:, 0:1]
    s_box = sel_ref[:, 1:2]
    s_cls = sel_ref[:, 2:3]
    deltas = sel_ref[:, 3:7]
    p_col = jax.lax.broadcasted_iota(i32, (K, 1), 0).astype(f32)
    real = p_col < float(PRE_NMS_TOP_N)
    q_idx = jax.lax.broadcasted_iota(i32, (K, K), 1)
    p_idx = jax.lax.broadcasted_iota(i32, (K, K), 0)

    def to_row(x):
        return jnp.swapaxes(x, 0, 1)

    by = jnp.floor(s_box * (1.0 / 64.0))
    bx = s_box - 64.0 * by
    ax = bx * float(STRIDE)
    ay = by * float(STRIDE)
    widths = jnp.full((K, 1), 64.0, f32)
    ctr_x = ax + 0.5
    ctr_y = ay + 0.5
    dx = deltas[:, 0:1] / 10.0
    dy = deltas[:, 1:2] / 10.0
    dw = jnp.minimum(deltas[:, 2:3] / 5.0, BBOX_XFORM_CLIP)
    dh = jnp.minimum(deltas[:, 3:4] / 5.0, BBOX_XFORM_CLIP)
    pcx = dx * widths + ctr_x
    pcy = dy * widths + ctr_y
    pw = jnp.exp(dw) * widths
    ph = jnp.exp(dh) * widths
    x1 = pcx - 0.5 * (pw - 1.0)
    y1 = pcy - 0.5 * (ph - 1.0)
    x2 = pcx + 0.5 * (pw - 1.0)
    y2 = pcy + 0.5 * (ph - 1.0)
    x1 = jnp.clip(x1, 0.0, 511.0)
    y1 = jnp.clip(y1, 0.0, 511.0)
    x2 = jnp.clip(x2, 0.0, 511.0)
    y2 = jnp.clip(y2, 0.0, 511.0)

    offv = s_cls * 513.0
    pad_off = jnp.where(real, 0.0, -1.0e7 - p_col * 1.0e3)
    nx1 = x1 + offv + pad_off
    ny1 = y1 + offv + pad_off
    nx2 = jnp.where(real, x2 + offv, nx1 - 1.0)
    ny2 = jnp.where(real, y2 + offv, ny1 - 1.0)

    areas = (nx2 - nx1 + 1.0) * (ny2 - ny1 + 1.0)
    xx1 = jnp.maximum(nx1, to_row(nx1))
    yy1 = jnp.maximum(ny1, to_row(ny1))
    xx2 = jnp.minimum(nx2, to_row(nx2))
    yy2 = jnp.minimum(ny2, to_row(ny2))
    w = jnp.maximum(xx2 - xx1 + 1.0, 0.0)
    h = jnp.maximum(yy2 - yy1 + 1.0, 0.0)
    inter = w * h
    iou = inter / (areas + to_row(areas) - inter)
    a_sup = ((iou > NMS_TH) & (p_idx < q_idx)).astype(f32)

    def cond(carry):
        return carry[1]

    def sweep(keep):
        sup = _dot(keep, a_sup, ((1,), (0,)), _DEF)
        return jnp.where(sup > 0.0, 0.0, 1.0)

    def body(carry):
        keep, _ = carry
        new = sweep(sweep(keep))
        return new, jnp.any(new != keep)

    keep0 = jnp.ones((1, K), f32)
    keep_row, _ = jax.lax.while_loop(cond, body, (keep0, jnp.bool_(True)))

    keep_col = jnp.swapaxes(keep_row, 0, 1)

    valid = keep_col * real.astype(f32)
    masked = jnp.where(valid > 0.0, s_score, -1.0e9)
    m_row = to_row(masked)
    better2 = (m_row > masked) | ((m_row == masked) & (q_idx < p_idx))
    rank2 = jnp.sum(better2.astype(f32), axis=1, keepdims=True)
    rank2_row = to_row(rank2)

    boxes4 = jnp.concatenate([x1, y1, x2, y2], axis=1) * valid
    score_out = jnp.where(valid > 0.0, s_score, 0.0)

    p128 = jax.lax.broadcasted_iota(i32, (128, K), 0).astype(f32)
    onehot_f = (rank2_row == p128).astype(f32)
    out4 = _dot(onehot_f, boxes4, ((1,), (0,)), _HI)
    outs = _dot(onehot_f, score_out, ((1,), (0,)), _HI)
    out_ref[...] = jnp.concatenate(
        [out4, outs, jnp.zeros((128, 3), f32)], axis=1)


def kernel(images, features, box_regression, centerness, box_cls,
           dot_product_logits, positive_map):
    del images, features, box_cls
    dp = dot_product_logits.reshape(N, 256)
    ctn = centerness.reshape(N, 1)
    br = jnp.transpose(box_regression.reshape(4, N), (1, 0))
    sel = pl.pallas_call(
        _select_body,
        out_shape=jax.ShapeDtypeStruct((K, 8), jnp.float32),
    )(dp, positive_map, ctn, br)
    out = pl.pallas_call(
        _nms_body,
        out_shape=jax.ShapeDtypeStruct((128, 8), jnp.float32),
    )(sel)
    return out[:MAX_DET, :5].reshape(1, MAX_DET, 5)


# speedup vs baseline: 38.9813x; 1.2092x over previous
import math

import jax
import jax.numpy as jnp
from jax.experimental import pallas as pl

STRIDE = 8
ANCHOR_SIZE = 64
PRE_NMS_THRESH = 0.05
PRE_NMS_TOP_N = 1000
NMS_TH = 0.6
MAX_DET = 100
BBOX_XFORM_CLIP = math.log(1000.0 / 16.0)

N = 4096
C = 80
K = 1024
NC_TOTAL = N * C

_HI = jax.lax.Precision.HIGHEST
_DEF = jax.lax.Precision.DEFAULT


def _dot(a, b, dims, prec):
    return jax.lax.dot_general(a, b, (dims, ((), ())), precision=prec,
                               preferred_element_type=jnp.float32)


def _select_body(dp_ref, pm_ref, ctn_ref, br_ref, sel_ref):
    f32, i32 = jnp.float32, jnp.int32

    pmv = pm_ref[...]
    pmn = pmv / jnp.maximum(jnp.sum(pmv, axis=-1, keepdims=True), 1e-6)
    logits = _dot(dp_ref[...], pmn, ((1,), (1,)), _DEF)
    ctn = jax.nn.sigmoid(ctn_ref[...])
    sc = jnp.sqrt(jax.nn.sigmoid(logits) * ctn)
    sc = jnp.where(sc > PRE_NMS_THRESH, sc, 0.0)
    bits = jax.lax.bitcast_convert_type(sc, i32)

    base = jnp.int32(0x3D4CCCCD)
    c0 = jnp.sum((bits > base).astype(i32))

    def bit_step(i, d):
        b = 25 - i
        d_try = jnp.bitwise_or(d, jax.lax.shift_left(jnp.int32(1), b))
        cnt = jnp.sum((bits > base + d_try).astype(i32))
        return jnp.where(cnt >= PRE_NMS_TOP_N, d_try, d)

    d = jax.lax.fori_loop(0, 26, bit_step, jnp.int32(0))
    v_bits = jnp.where(c0 >= PRE_NMS_TOP_N, base + d + 1, jnp.int32(0))
    m_strict = bits > v_bits
    m_eq = bits == v_bits
    n_strict = jnp.sum(m_strict.astype(i32))
    needed_m1 = (PRE_NMS_TOP_N - 1 - n_strict).astype(f32)

    lane_q = jax.lax.broadcasted_iota(i32, (128, 128), 0)
    lane_p = jax.lax.broadcasted_iota(i32, (128, 128), 1)
    u_le = (lane_q <= lane_p).astype(f32)
    c32_q = jax.lax.broadcasted_iota(i32, (32, 32), 1)
    c32_p = jax.lax.broadcasted_iota(i32, (32, 32), 0)
    l32 = (c32_q < c32_p).astype(f32)
    c_lt = jax.lax.broadcasted_iota(i32, (C, C), 0)
    c_gt = jax.lax.broadcasted_iota(i32, (C, C), 1)
    lstrict_c = (c_lt < c_gt).astype(f32)

    def excl_rowpref(cnt_col):
        c2d = cnt_col.reshape(32, 128)
        incl = _dot(c2d, u_le, ((1,), (0,)), _DEF)
        off = _dot(l32, incl[:, 127:128], ((1,), (0,)), _HI)
        return off + incl - c2d

    m_eq_f = m_eq.astype(f32)
    rowpref_eq = excl_rowpref(jnp.sum(m_eq_f, axis=1, keepdims=True))
    mrow = rowpref_eq <= needed_m1
    r_star = jnp.sum(mrow.astype(f32)) - 1.0
    prefref = jnp.max(jnp.where(mrow, rowpref_eq, -1.0))
    q_star = needed_m1 - prefref
    wpref_eq = _dot(m_eq_f, lstrict_c, ((1,), (0,)), _DEF)
    row_f = jax.lax.broadcasted_iota(i32, (N, C), 0).astype(f32)
    self_f = (m_strict | (m_eq & ((row_f < r_star)
                                  | ((row_f == r_star)
                                     & (wpref_eq <= q_star))))).astype(f32)

    cnt_row = jnp.sum(self_f, axis=1, keepdims=True)
    rowpref = excl_rowpref(cnt_row)

    p_col = jax.lax.broadcasted_iota(i32, (K, 1), 0).astype(f32)
    r_cnt = jnp.zeros((K, 1), f32)
    pref_r = jnp.full((K, 1), -1.0, f32)
    for i in range(32):
        row = rowpref[i:i + 1, :]
        m = row <= p_col
        r_cnt = r_cnt + jnp.sum(m.astype(f32), axis=1, keepdims=True)
        pref_r = jnp.maximum(
            pref_r, jnp.max(jnp.where(m, row, -1.0), axis=1, keepdims=True))
    r_p = r_cnt - 1.0
    r_p_i = r_p.astype(i32)

    m16 = jnp.int32(-65536)
    def planes(x):
        xhi = jax.lax.bitcast_convert_type(
            jnp.bitwise_and(jax.lax.bitcast_convert_type(x, i32), m16), f32)
        xlo = x - xhi
        xmid = jax.lax.bitcast_convert_type(
            jnp.bitwise_and(jax.lax.bitcast_convert_type(xlo, i32), m16), f32)
        return xhi, xmid, xlo - xmid

    schi, scmid, sclo = planes(sc)
    brv = br_ref[...]
    brhi, brmid, brlo = planes(brv)
    payload = jnp.concatenate(
        [self_f, schi, scmid, sclo, brhi, brmid, brlo], axis=1)

    gath = jnp.zeros((K, 332), f32)
    for i in range(4):
        ch = jax.lax.broadcasted_iota(i32, (K, K), 1) + (i * K)
        oh = (ch == r_p_i).astype(f32)
        gath = gath + _dot(oh, payload[i * K:(i + 1) * K, :], ((1,), (0,)), _DEF)
    selrows = gath[:, 0:C]
    scrows = (gath[:, C:2 * C] + gath[:, 2 * C:3 * C]) + gath[:, 3 * C:4 * C]
    br_p = (gath[:, 4 * C:4 * C + 4] + gath[:, 4 * C + 4:4 * C + 8]) \
        + gath[:, 4 * C + 8:4 * C + 12]

    q_p = p_col - pref_r
    c_lt = jax.lax.broadcasted_iota(i32, (C, C), 0)
    c_gt = jax.lax.broadcasted_iota(i32, (C, C), 1)
    lstrict_c = (c_lt < c_gt).astype(f32)
    wpref = _dot(selrows, lstrict_c, ((1,), (0,)), _DEF)
    colmatch = selrows * (wpref == q_p).astype(f32)
    col_iota = jax.lax.broadcasted_iota(i32, (K, C), 1).astype(f32)
    c_p = jnp.sum(colmatch * col_iota, axis=1, keepdims=True)
    score_p = jnp.sum(colmatch * scrows, axis=1, keepdims=True)

    real = p_col < float(PRE_NMS_TOP_N)
    score_p = jnp.where(real, score_p, 0.0)
    box_p = jnp.where(real, r_p, 0.0)
    cls_p = jnp.where(real, c_p, 0.0)
    br_p = jnp.where(real, br_p, 0.0)

    def to_row(x):
        return jnp.swapaxes(x, 0, 1)

    k_col = jnp.where(real, score_p, -1.0)
    k_row = to_row(k_col)
    q_idx = jax.lax.broadcasted_iota(i32, (K, K), 1)
    p_idx = jax.lax.broadcasted_iota(i32, (K, K), 0)
    better = (k_row > k_col) | ((k_row == k_col) & (q_idx < p_idx))
    rank = jnp.sum(better.astype(f32), axis=1, keepdims=True)
    rank_row = to_row(rank)
    onehot_s = (rank_row == p_col).astype(f32)

    pay2 = jnp.concatenate([score_p, box_p, cls_p, br_p], axis=1)
    s_pay = _dot(onehot_s, pay2, ((1,), (0,)), _HI)
    sel_ref[...] = jnp.concatenate([s_pay, jnp.zeros((K, 1), f32)], axis=1)


def _nms_body(sel_ref, out_ref):
    f32, i32 = jnp.float32, jnp.int32
    s_score = sel_ref[:, 0:1]
    s_box = sel_ref[:, 1:2]
    s_cls = sel_ref[:, 2:3]
    deltas = sel_ref[:, 3:7]
    p_col = jax.lax.broadcasted_iota(i32, (K, 1), 0).astype(f32)
    real = p_col < float(PRE_NMS_TOP_N)
    q_idx = jax.lax.broadcasted_iota(i32, (K, K), 1)
    p_idx = jax.lax.broadcasted_iota(i32, (K, K), 0)

    def to_row(x):
        return jnp.swapaxes(x, 0, 1)

    by = jnp.floor(s_box * (1.0 / 64.0))
    bx = s_box - 64.0 * by
    ax = bx * float(STRIDE)
    ay = by * float(STRIDE)
    widths = jnp.full((K, 1), 64.0, f32)
    ctr_x = ax + 0.5
    ctr_y = ay + 0.5
    dx = deltas[:, 0:1] / 10.0
    dy = deltas[:, 1:2] / 10.0
    dw = jnp.minimum(deltas[:, 2:3] / 5.0, BBOX_XFORM_CLIP)
    dh = jnp.minimum(deltas[:, 3:4] / 5.0, BBOX_XFORM_CLIP)
    pcx = dx * widths + ctr_x
    pcy = dy * widths + ctr_y
    pw = jnp.exp(dw) * widths
    ph = jnp.exp(dh) * widths
    x1 = pcx - 0.5 * (pw - 1.0)
    y1 = pcy - 0.5 * (ph - 1.0)
    x2 = pcx + 0.5 * (pw - 1.0)
    y2 = pcy + 0.5 * (ph - 1.0)
    x1 = jnp.clip(x1, 0.0, 511.0)
    y1 = jnp.clip(y1, 0.0, 511.0)
    x2 = jnp.clip(x2, 0.0, 511.0)
    y2 = jnp.clip(y2, 0.0, 511.0)

    offv = s_cls * 513.0
    pad_off = jnp.where(real, 0.0, -1.0e7 - p_col * 1.0e3)
    nx1 = x1 + offv + pad_off
    ny1 = y1 + offv + pad_off
    nx2 = jnp.where(real, x2 + offv, nx1 - 1.0)
    ny2 = jnp.where(real, y2 + offv, ny1 - 1.0)

    areas = (nx2 - nx1 + 1.0) * (ny2 - ny1 + 1.0)
    xx1 = jnp.maximum(nx1, to_row(nx1))
    yy1 = jnp.maximum(ny1, to_row(ny1))
    xx2 = jnp.minimum(nx2, to_row(nx2))
    yy2 = jnp.minimum(ny2, to_row(ny2))
    w = jnp.maximum(xx2 - xx1 + 1.0, 0.0)
    h = jnp.maximum(yy2 - yy1 + 1.0, 0.0)
    inter = w * h
    iou = inter / (areas + to_row(areas) - inter)
    a_sup = ((iou > NMS_TH) & (p_idx < q_idx)).astype(f32)

    def cond(carry):
        return carry[1]

    def sweep(keep):
        sup = _dot(keep, a_sup, ((1,), (0,)), _DEF)
        return jnp.where(sup > 0.0, 0.0, 1.0)

    def body(carry):
        keep, _ = carry
        new = sweep(sweep(keep))
        return new, jnp.any(new != keep)

    keep0 = jnp.ones((1, K), f32)
    keep_row, _ = jax.lax.while_loop(cond, body, (keep0, jnp.bool_(True)))

    keep_col = jnp.swapaxes(keep_row, 0, 1)

    valid = keep_col * real.astype(f32)
    masked = jnp.where(valid > 0.0, s_score, -1.0e9)
    m_row = to_row(masked)
    better2 = (m_row > masked) | ((m_row == masked) & (q_idx < p_idx))
    rank2 = jnp.sum(better2.astype(f32), axis=1, keepdims=True)
    rank2_row = to_row(rank2)

    boxes4 = jnp.concatenate([x1, y1, x2, y2], axis=1) * valid
    score_out = jnp.where(valid > 0.0, s_score, 0.0)

    p128 = jax.lax.broadcasted_iota(i32, (128, K), 0).astype(f32)
    onehot_f = (rank2_row == p128).astype(f32)
    out4 = _dot(onehot_f, boxes4, ((1,), (0,)), _HI)
    outs = _dot(onehot_f, score_out, ((1,), (0,)), _HI)
    out_ref[...] = jnp.concatenate(
        [out4, outs, jnp.zeros((128, 3), f32)], axis=1)


def kernel(images, features, box_regression, centerness, box_cls,
           dot_product_logits, positive_map):
    del images, features, box_cls
    dp = dot_product_logits.reshape(N, 256)
    ctn = centerness.reshape(N, 1)
    br = jnp.transpose(box_regression.reshape(4, N), (1, 0))
    sel = pl.pallas_call(
        _select_body,
        out_shape=jax.ShapeDtypeStruct((K, 8), jnp.float32),
    )(dp, positive_map, ctn, br)
    out = pl.pallas_call(
        _nms_body,
        out_shape=jax.ShapeDtypeStruct((128, 8), jnp.float32),
    )(sel)
    return out[:MAX_DET, :5].reshape(1, MAX_DET, 5)
